# baseline (speedup 1.0000x reference)
import functools

import jax
import jax.numpy as jnp
from jax import lax
from jax.experimental import pallas as pl
from jax.experimental.pallas import tpu as pltpu
from jax.experimental.pallas import tpu_sc as plsc

_NBUF = 8
_AHEAD = 6


def _make_sc_kernel(B, S, V, D, NC=2, NS=16, L=16, interpret=False):
  NW = NC * NS
  SPW = B // NW
  mesh = plsc.VectorSubcoreMesh(core_axis_name="c", subcore_axis_name="s",
                                num_cores=NC, num_subcores=NS)

  @functools.partial(
      pl.kernel,
      out_type=jax.ShapeDtypeStruct((B, S, D), jnp.float32),
      mesh=mesh,
      scratch_types=[
          pltpu.VMEM((S, D), jnp.float32),
          pltpu.VMEM((S, SPW), jnp.int32),
          pltpu.VMEM((_NBUF, SPW, D), jnp.float32),
          pltpu.SemaphoreType.DMA((_NBUF,)),
          pltpu.SemaphoreType.DMA((_NBUF,)),
      ],
      interpret=interpret,
  )
  def k(x_hbm, tok_hbm, pos_hbm, out_hbm, pos_v, idx_v, buf_v, gsem, osem):
    wid = lax.axis_index("s") * NC + lax.axis_index("c")
    pltpu.sync_copy(x_hbm.at[wid], idx_v)
    pltpu.sync_copy(pos_hbm, pos_v)
    seq0 = wid * SPW

    def start_gather(s, b):
      hh = SPW // 2
      pltpu.async_copy(tok_hbm.at[idx_v.at[s, pl.ds(0, hh)]],
                       buf_v.at[b, pl.ds(0, hh)], gsem.at[b])
      pltpu.async_copy(tok_hbm.at[idx_v.at[s, pl.ds(hh, hh)]],
                       buf_v.at[b, pl.ds(hh, hh)], gsem.at[b])

    def wait_gather(b):
      pltpu.make_async_copy(tok_hbm.at[pl.ds(0, SPW)], buf_v.at[b],
                            gsem.at[b]).wait()

    def start_scatter(s, b):
      pltpu.async_copy(buf_v.at[b], out_hbm.at[pl.ds(seq0, SPW), s],
                       osem.at[b])

    def wait_scatter(b):
      pltpu.make_async_copy(buf_v.at[b], out_hbm.at[pl.ds(0, SPW), 0],
                            osem.at[b]).wait()

    for s in range(_AHEAD):
      start_gather(s, s)

    def body(g, carry):
      for bb in range(_NBUF):
        s = g * _NBUF + bb
        @pl.when(s + _AHEAD < S)
        def _():
          @pl.when(s + _AHEAD >= _NBUF)
          def _():
            wait_scatter((s + _AHEAD) % _NBUF)
          start_gather(s + _AHEAD, (s + _AHEAD) % _NBUF)
        wait_gather(bb)

        pos_regs = tuple(pos_v[s, pl.ds(l * L, L)] for l in range(D // L))

        def add_body(j, pregs):
          for l in range(D // L):
            sl = pl.ds(l * L, L)
            buf_v[bb, j, sl] = buf_v[bb, j, sl] + pregs[l]
          return pregs

        lax.fori_loop(0, SPW, add_body, pos_regs)
        start_scatter(s, bb)
      return carry

    lax.fori_loop(0, S // _NBUF, body, 0)
    for b in range(_NBUF):
      wait_scatter(b)

  return k


def kernel(x, token_table, pos_table):
  B, S = x.shape
  V, D = token_table.shape
  info = plsc.get_sparse_core_info()
  NC, NS, L = info.num_cores, info.num_subcores, info.num_lanes
  NW = NC * NS
  xw = jnp.swapaxes(x.astype(jnp.int32).reshape(NW, B // NW, S), 1, 2)
  k = _make_sc_kernel(B, S, V, D, NC=NC, NS=NS, L=L)
  return k(xw, token_table, pos_table)


# speedup vs baseline: 1.0011x; 1.0011x over previous
import functools

import jax
import jax.numpy as jnp
from jax import lax
from jax.experimental import pallas as pl
from jax.experimental.pallas import tpu as pltpu
from jax.experimental.pallas import tpu_sc as plsc

_NBUF = 8
_AHEAD = 6


def _make_sc_kernel(B, S, V, D, NC=2, NS=16, L=16, interpret=False):
  NW = NC * NS
  SPW = B // NW
  mesh = plsc.VectorSubcoreMesh(core_axis_name="c", subcore_axis_name="s",
                                num_cores=NC, num_subcores=NS)

  @functools.partial(
      pl.kernel,
      out_type=jax.ShapeDtypeStruct((B, S, D), jnp.float32),
      mesh=mesh,
      scratch_types=[
          pltpu.VMEM((S, D), jnp.float32),
          pltpu.VMEM((S, SPW), jnp.int32),
          pltpu.VMEM((_NBUF, SPW, D), jnp.float32),
          pltpu.SemaphoreType.DMA((_NBUF,)),
          pltpu.SemaphoreType.DMA((_NBUF,)),
          pltpu.SemaphoreType.DMA,
      ],
      interpret=interpret,
  )
  def k(x_hbm, tok_hbm, pos_hbm, out_hbm, pos_v, idx_v, buf_v, gsem, osem, psem):
    wid = lax.axis_index("s") * NC + lax.axis_index("c")
    pltpu.sync_copy(x_hbm.at[wid], idx_v)
    pos_cp = pltpu.async_copy(pos_hbm, pos_v, psem)
    seq0 = wid * SPW

    def start_gather(s, b):
      pltpu.async_copy(tok_hbm.at[idx_v.at[s]], buf_v.at[b], gsem.at[b])

    def wait_gather(b):
      pltpu.make_async_copy(tok_hbm.at[pl.ds(0, SPW)], buf_v.at[b],
                            gsem.at[b]).wait()

    def start_scatter(s, b):
      pltpu.async_copy(buf_v.at[b], out_hbm.at[pl.ds(seq0, SPW), s],
                       osem.at[b])

    def wait_scatter(b):
      pltpu.make_async_copy(buf_v.at[b], out_hbm.at[pl.ds(0, SPW), 0],
                            osem.at[b]).wait()

    for s in range(_AHEAD):
      start_gather(s, s)
    pos_cp.wait()

    def body(g, carry):
      for bb in range(_NBUF):
        s = g * _NBUF + bb
        @pl.when(s + _AHEAD < S)
        def _():
          @pl.when(s + _AHEAD >= _NBUF)
          def _():
            wait_scatter((s + _AHEAD) % _NBUF)
          start_gather(s + _AHEAD, (s + _AHEAD) % _NBUF)
        wait_gather(bb)

        pos_regs = tuple(pos_v[s, pl.ds(l * L, L)] for l in range(D // L))

        def add_body(j, pregs):
          for l in range(D // L):
            sl = pl.ds(l * L, L)
            buf_v[bb, j, sl] = buf_v[bb, j, sl] + pregs[l]
          return pregs

        lax.fori_loop(0, SPW, add_body, pos_regs)
        start_scatter(s, bb)
      return carry

    lax.fori_loop(0, S // _NBUF, body, 0)
    for b in range(_NBUF):
      wait_scatter(b)

  return k


def kernel(x, token_table, pos_table):
  B, S = x.shape
  V, D = token_table.shape
  info = plsc.get_sparse_core_info()
  NC, NS, L = info.num_cores, info.num_subcores, info.num_lanes
  NW = NC * NS
  xw = jnp.swapaxes(x.astype(jnp.int32).reshape(NW, B // NW, S), 1, 2)
  k = _make_sc_kernel(B, S, V, D, NC=NC, NS=NS, L=L)
  return k(xw, token_table, pos_table)
